# baseline (speedup 1.0000x reference)
import functools

import jax
import jax.numpy as jnp
from jax import lax
from jax.experimental import pallas as pl
from jax.experimental.pallas import tpu as pltpu
from jax.experimental.pallas import tpu_sc as plsc

B = 16384
NUM_STAGES = 16
NUM_VIEWS = 8
NUM_COMBOS = NUM_STAGES * NUM_VIEWS
EMBED = 1024
HIDDEN = 4096
EXPERTS = 64


K_STEPS = 4
HBLK = HIDDEN // K_STEPS


def _table_body(st_ref, vt_ref, w1_ref, b1_ref, w2t_ref, b2_ref,
                probs_t_ref, sel_ref, ct_ref, h_ref):
    k = pl.program_id(0)

    @pl.when(k == 0)
    def _build_combos():
        sbc = jnp.broadcast_to(st_ref[...][:, None, :],
                               (NUM_STAGES, NUM_VIEWS, EMBED))
        vbc = jnp.broadcast_to(vt_ref[...][None, :, :],
                               (NUM_STAGES, NUM_VIEWS, EMBED))
        ct_ref[:, :EMBED] = sbc.reshape(NUM_COMBOS, EMBED)
        ct_ref[:, EMBED:] = vbc.reshape(NUM_COMBOS, EMBED)

    h = jnp.dot(ct_ref[...], w1_ref[...], preferred_element_type=jnp.float32)
    h_ref[:, pl.ds(k * HBLK, HBLK)] = jnp.maximum(h + b1_ref[...], 0.0)

    @pl.when(k == K_STEPS - 1)
    def _finish():
        logits = lax.dot_general(
            h_ref[...], w2t_ref[...], (((1,), (1,)), ((), ())),
            preferred_element_type=jnp.float32) + b2_ref[...]
        m = jnp.max(logits, axis=-1, keepdims=True)
        e = jnp.exp(logits - m)
        p = e / jnp.sum(e, axis=-1, keepdims=True)
        probs_t_ref[...] = jnp.transpose(p)
        pmax = jnp.max(p, axis=-1, keepdims=True)
        col = lax.broadcasted_iota(jnp.int32, (NUM_COMBOS, EXPERTS), 1)
        cand = jnp.where(p == pmax, col, EXPERTS)
        sel_ref[...] = jnp.min(cand, axis=-1)


_table_call = pl.pallas_call(
    _table_body,
    grid=(K_STEPS,),
    in_specs=[
        pl.BlockSpec((NUM_STAGES, EMBED), lambda k: (0, 0)),
        pl.BlockSpec((NUM_VIEWS, EMBED), lambda k: (0, 0)),
        pl.BlockSpec((2 * EMBED, HBLK), lambda k: (0, k)),
        pl.BlockSpec((1, HBLK), lambda k: (0, k)),
        pl.BlockSpec((EXPERTS, HIDDEN), lambda k: (0, 0)),
        pl.BlockSpec((1, EXPERTS), lambda k: (0, 0)),
    ],
    out_specs=[
        pl.BlockSpec((EXPERTS, NUM_COMBOS), lambda k: (0, 0)),
        pl.BlockSpec((NUM_COMBOS,), lambda k: (0,)),
    ],
    out_shape=[
        jax.ShapeDtypeStruct((EXPERTS, NUM_COMBOS), jnp.float32),
        jax.ShapeDtypeStruct((NUM_COMBOS,), jnp.int32),
    ],
    scratch_shapes=[
        pltpu.VMEM((NUM_COMBOS, 2 * EMBED), jnp.float32),
        pltpu.VMEM((NUM_COMBOS, HIDDEN), jnp.float32),
    ],
)


NC, NS, L = 2, 16, 16
NW = NC * NS
BPW = B // NW
GROUPS = BPW // L


@functools.partial(
    pl.kernel,
    out_type=(jax.ShapeDtypeStruct((EXPERTS, B), jnp.float32),
              jax.ShapeDtypeStruct((B,), jnp.int32)),
    mesh=plsc.VectorSubcoreMesh(core_axis_name="c", subcore_axis_name="s"),
    scratch_types=[
        pltpu.VMEM((BPW,), jnp.int32),
        pltpu.VMEM((BPW,), jnp.int32),
        pltpu.VMEM((BPW,), jnp.int32),
        pltpu.VMEM((NUM_COMBOS,), jnp.int32),
        pltpu.VMEM((BPW,), jnp.int32),
        pltpu.VMEM((EXPERTS, NUM_COMBOS), jnp.float32),
        pltpu.VMEM((EXPERTS, BPW), jnp.float32),
        pltpu.SemaphoreType.DMA,
        pltpu.SemaphoreType.DMA,
    ],
    compiler_params=pltpu.CompilerParams(needs_layout_passes=False,
                                         use_tc_tiling_on_sc=True),
)
def _lookup_call(ptabt_hbm, asel_hbm, sid_hbm, vid_hbm, probs_hbm, sel_hbm,
                 s_v, v_v, idx_v, asel_v, sel_v, tab_v, rows_v, sem, wsem):
    wid = lax.axis_index("s") * NC + lax.axis_index("c")
    base = wid * BPW
    in0 = pltpu.async_copy(sid_hbm.at[pl.ds(base, BPW)], s_v, sem)
    in1 = pltpu.async_copy(vid_hbm.at[pl.ds(base, BPW)], v_v, sem)
    in2 = pltpu.async_copy(asel_hbm, asel_v, sem)
    in3 = pltpu.async_copy(ptabt_hbm, tab_v, sem)
    in0.wait()
    in1.wait()
    in2.wait()
    in3.wait()
    @plsc.parallel_loop(0, GROUPS, unroll=4)
    def idx_body(g):
        s16 = s_v[pl.ds(g * L, L)]
        v16 = v_v[pl.ds(g * L, L)]
        idx16 = s16 * NUM_VIEWS + v16
        idx_v[pl.ds(g * L, L)] = idx16
        sel_v[pl.ds(g * L, L)] = plsc.load_gather(asel_v, [idx16])

    def run_half(e_lo, e_hi):
        @plsc.parallel_loop(0, GROUPS, unroll=2)
        def group_body(g):
            idx16 = idx_v[pl.ds(g * L, L)]
            for e in range(e_lo, e_hi):
                e16 = jnp.full((L,), e, jnp.int32)
                rows_v[e, pl.ds(g * L, L)] = plsc.load_gather(tab_v,
                                                              [e16, idx16])

    HALF = EXPERTS // 2
    run_half(0, HALF)
    w0 = pltpu.async_copy(rows_v.at[pl.ds(0, HALF)],
                          probs_hbm.at[pl.ds(0, HALF), pl.ds(base, BPW)], wsem)
    w1 = pltpu.async_copy(sel_v, sel_hbm.at[pl.ds(base, BPW)], wsem)
    run_half(HALF, EXPERTS)
    w2 = pltpu.async_copy(rows_v.at[pl.ds(HALF, HALF)],
                          probs_hbm.at[pl.ds(HALF, HALF), pl.ds(base, BPW)],
                          wsem)
    w0.wait()
    w1.wait()
    w2.wait()


def kernel(stage_ids, view_ids, stage_table, view_table, W1, b1, W2, b2):
    probs_t_tab, argsel_tab = _table_call(
        stage_table, view_table, W1,
        b1.reshape(1, HIDDEN), jnp.transpose(W2), b2.reshape(1, EXPERTS))
    probs_t, selected = _lookup_call(
        probs_t_tab, argsel_tab,
        stage_ids.astype(jnp.int32), view_ids.astype(jnp.int32))
    return jnp.transpose(probs_t), selected
